# Initial kernel scaffold; baseline (speedup 1.0000x reference)
import jax
import jax.numpy as jnp
from jax.experimental import pallas as pl


def kernel(x, edge_index, batch, W0, b0, W1, b1, W2, b2, Wlin, blin):
    raise NotImplementedError("write your pallas kernel here")


# speedup vs baseline: 7.5754x; 7.5754x over previous
import functools

import jax
import jax.numpy as jnp
from jax import lax
from jax.experimental import pallas as pl
from jax.experimental.pallas import tpu as pltpu
from jax.experimental.pallas import tpu_sc as plsc

N_NODES = 10000
D_FEAT = 128
HIDDEN = 128
N_GRAPHS = 64
N_CLASSES = 10
N_EDGES = 320000

NCORES = 2
NSUB = 16
NW = NCORES * NSUB
K = 128
CPT = 80
CH_PAD = CPT * NW
E_PAD = CH_PAD * K
NPAD = 10240
ZPT = NPAD // NSUB

_MESH = plsc.VectorSubcoreMesh(
    core_axis_name="c", subcore_axis_name="s", num_cores=NCORES,
    num_subcores=NSUB)


@functools.partial(
    pl.kernel,
    out_type=jax.ShapeDtypeStruct((NW * NPAD,), jnp.float32),
    mesh=_MESH,
    scratch_types=[
        pltpu.VMEM((CPT, K), jnp.int32),
        pltpu.VMEM((NPAD,), jnp.float32),
    ],
    compiler_params=pltpu.CompilerParams(needs_layout_passes=False),
)
def _sc_deg(dst_hbm, out_hbm, dst_v, deg_v):
    c = lax.axis_index("c")
    s = lax.axis_index("s")
    w = c * NSUB + s
    pltpu.sync_copy(dst_hbm.at[pl.ds(w * CPT, CPT)], dst_v)

    def zbody(i, carry):
        deg_v[pl.ds(i * 16, 16)] = jnp.zeros((16,), jnp.float32)
        return carry

    lax.fori_loop(0, NPAD // 16, zbody, 0)
    ones = jnp.ones((16,), jnp.float32)

    def body(i, carry):
        d = dst_v[i // 8, pl.ds((i % 8) * 16, 16)]
        plsc.addupdate_scatter(deg_v, [d], ones)
        return carry

    lax.fori_loop(0, CPT * 8, body, 0)
    pltpu.sync_copy(deg_v, out_hbm.at[pl.ds(w * NPAD, NPAD)])


@functools.partial(
    pl.kernel,
    out_type=jax.ShapeDtypeStruct((NCORES, NPAD, HIDDEN), jnp.float32),
    mesh=_MESH,
    scratch_types=[
        pltpu.VMEM((CPT, K), jnp.int32),
        pltpu.VMEM((CPT, K), jnp.int32),
        pltpu.VMEM((K, HIDDEN), jnp.float32),
        pltpu.VMEM_SHARED((NPAD, HIDDEN), jnp.float32),
        pltpu.SemaphoreType.DMA,
    ],
)
def _sc_prop(g_hbm, src_hbm, dst_hbm, zeros_hbm, out_hbm,
             src_v, dst_v, rows_v, acc, sem):
    c = lax.axis_index("c")
    s = lax.axis_index("s")
    w = c * NSUB + s
    pltpu.sync_copy(src_hbm.at[pl.ds(w * CPT, CPT)], src_v)
    pltpu.sync_copy(dst_hbm.at[pl.ds(w * CPT, CPT)], dst_v)
    pltpu.sync_copy(zeros_hbm, acc.at[pl.ds(s * ZPT, ZPT)])
    plsc.subcore_barrier()

    def body(i, carry):
        pltpu.async_copy(g_hbm.at[src_v.at[i]], rows_v, sem).wait()
        pltpu.sync_copy(rows_v, acc.at[dst_v.at[i]], add=True)
        return carry

    lax.fori_loop(0, CPT, body, 0)
    plsc.subcore_barrier()
    pltpu.sync_copy(acc.at[pl.ds(s * ZPT, ZPT)],
                    out_hbm.at[c, pl.ds(s * ZPT, ZPT)])


_RB = 1000
_GRID = N_NODES // _RB
_RB1 = 2048
_GRID1 = NPAD // _RB1


def _tc_first_body(deg_ref, x_ref, w_ref, dis_ref, g_ref):
    deg = jnp.sum(deg_ref[...], axis=0)[:, None] + 1.0
    dis = lax.rsqrt(jnp.maximum(deg, 1e-12))
    dis_ref[...] = dis
    g_ref[...] = dis * jnp.dot(x_ref[...], w_ref[...],
                               preferred_element_type=jnp.float32)


def _tc_first(deg_parts, x, W0):
    return pl.pallas_call(
        _tc_first_body,
        grid=(_GRID1,),
        in_specs=[
            pl.BlockSpec((NW, _RB1), lambda i: (0, i)),
            pl.BlockSpec((_RB1, D_FEAT), lambda i: (i, 0)),
            pl.BlockSpec((D_FEAT, HIDDEN), lambda i: (0, 0)),
        ],
        out_specs=[
            pl.BlockSpec((_RB1, 1), lambda i: (i, 0)),
            pl.BlockSpec((_RB1, HIDDEN), lambda i: (i, 0)),
        ],
        out_shape=[
            jax.ShapeDtypeStruct((N_NODES, 1), jnp.float32),
            jax.ShapeDtypeStruct((N_NODES, HIDDEN), jnp.float32),
        ],
    )(deg_parts, x, W0)


def _tc_mid_body(z_ref, g_ref, dis_ref, b_ref, w_ref, out_ref):
    dis = dis_ref[...]
    h = dis * (z_ref[0] + z_ref[1] + g_ref[...]) + b_ref[...]
    h = jnp.maximum(h, 0.0)
    out_ref[...] = dis * jnp.dot(h, w_ref[...],
                                 preferred_element_type=jnp.float32)


def _tc_mid(z, g, dis, b, W):
    return pl.pallas_call(
        _tc_mid_body,
        grid=(_GRID,),
        in_specs=[
            pl.BlockSpec((NCORES, _RB, HIDDEN), lambda i: (0, i, 0)),
            pl.BlockSpec((_RB, HIDDEN), lambda i: (i, 0)),
            pl.BlockSpec((_RB, 1), lambda i: (i, 0)),
            pl.BlockSpec((1, HIDDEN), lambda i: (0, 0)),
            pl.BlockSpec((HIDDEN, HIDDEN), lambda i: (0, 0)),
        ],
        out_specs=pl.BlockSpec((_RB, HIDDEN), lambda i: (i, 0)),
        out_shape=jax.ShapeDtypeStruct((N_NODES, HIDDEN), jnp.float32),
    )(z, g, dis, b, W)


def _tc_final_body(z_ref, g_ref, dis_ref, b_ref, batch_ref, wl_ref, bl_ref,
                   out_ref, accs, acccnt):
    i = pl.program_id(0)

    @pl.when(i == 0)
    def _():
        accs[...] = jnp.zeros_like(accs)
        acccnt[...] = jnp.zeros_like(acccnt)

    dis = dis_ref[...]
    h = dis * (z_ref[0] + z_ref[1] + g_ref[...]) + b_ref[...]
    h = jnp.maximum(h, 0.0)
    bidx = batch_ref[0, 0, :]
    gids = lax.broadcasted_iota(jnp.int32, (N_GRAPHS, _RB), 0)
    onehot = (gids == jnp.broadcast_to(bidx[None, :], (N_GRAPHS, _RB))
              ).astype(jnp.float32)
    accs[...] += jnp.dot(onehot, h, preferred_element_type=jnp.float32)
    acccnt[...] += jnp.sum(onehot, axis=1, keepdims=True)

    @pl.when(i == _GRID - 1)
    def _():
        pooled = accs[...] / jnp.maximum(acccnt[...], 1.0)
        out_ref[...] = jnp.dot(pooled, wl_ref[...],
                               preferred_element_type=jnp.float32) + bl_ref[...]


def _tc_final(z, g, dis, b, batchr, Wlin, blin):
    return pl.pallas_call(
        _tc_final_body,
        grid=(_GRID,),
        in_specs=[
            pl.BlockSpec((NCORES, _RB, HIDDEN), lambda i: (0, i, 0)),
            pl.BlockSpec((_RB, HIDDEN), lambda i: (i, 0)),
            pl.BlockSpec((_RB, 1), lambda i: (i, 0)),
            pl.BlockSpec((1, HIDDEN), lambda i: (0, 0)),
            pl.BlockSpec((1, 1, _RB), lambda i: (i, 0, 0)),
            pl.BlockSpec((HIDDEN, N_CLASSES), lambda i: (0, 0)),
            pl.BlockSpec((1, N_CLASSES), lambda i: (0, 0)),
        ],
        out_specs=pl.BlockSpec((N_GRAPHS, N_CLASSES), lambda i: (0, 0)),
        out_shape=jax.ShapeDtypeStruct((N_GRAPHS, N_CLASSES), jnp.float32),
        scratch_shapes=[
            pltpu.VMEM((N_GRAPHS, HIDDEN), jnp.float32),
            pltpu.VMEM((N_GRAPHS, 1), jnp.float32),
        ],
    )(z, g, dis, b, batchr, Wlin, blin)


def kernel(x, edge_index, batch, W0, b0, W1, b1, W2, b2, Wlin, blin):
    pad = E_PAD - N_EDGES
    src2d = jnp.concatenate(
        [edge_index[0], jnp.zeros((pad,), jnp.int32)]).reshape(CH_PAD, K)
    dst2d = jnp.concatenate(
        [edge_index[1], jnp.full((pad,), N_NODES, jnp.int32)]).reshape(CH_PAD, K)

    zrows = jnp.zeros((ZPT, HIDDEN), jnp.float32)
    batchr = batch.reshape(_GRID, 1, _RB)
    b0r = b0.reshape(1, HIDDEN)
    b1r = b1.reshape(1, HIDDEN)
    b2r = b2.reshape(1, HIDDEN)
    blr = blin.reshape(1, N_CLASSES)

    deg_parts = _sc_deg(dst2d).reshape(NW, NPAD)
    dis, g0 = _tc_first(deg_parts, x, W0)
    z0 = _sc_prop(g0, src2d, dst2d, zrows)
    g1 = _tc_mid(z0, g0, dis, b0r, W1)
    z1 = _sc_prop(g1, src2d, dst2d, zrows)
    g2 = _tc_mid(z1, g1, dis, b1r, W2)
    z2 = _sc_prop(g2, src2d, dst2d, zrows)
    return _tc_final(z2, g2, dis, b2r, batchr, Wlin, blr)


# speedup vs baseline: 7.6082x; 1.0043x over previous
import functools

import jax
import jax.numpy as jnp
from jax import lax
from jax.experimental import pallas as pl
from jax.experimental.pallas import tpu as pltpu
from jax.experimental.pallas import tpu_sc as plsc

N_NODES = 10000
D_FEAT = 128
HIDDEN = 128
N_GRAPHS = 64
N_CLASSES = 10
N_EDGES = 320000

NCORES = 2
NSUB = 16
NW = NCORES * NSUB
K = 128
CPT = 80
CH_PAD = CPT * NW
E_PAD = CH_PAD * K
NPAD = 10240
ZPT = NPAD // NSUB

_MESH = plsc.VectorSubcoreMesh(
    core_axis_name="c", subcore_axis_name="s", num_cores=NCORES,
    num_subcores=NSUB)


@functools.partial(
    pl.kernel,
    out_type=jax.ShapeDtypeStruct((NW * NPAD,), jnp.float32),
    mesh=_MESH,
    scratch_types=[
        pltpu.VMEM((CPT, K), jnp.int32),
        pltpu.VMEM((NPAD,), jnp.float32),
    ],
    compiler_params=pltpu.CompilerParams(needs_layout_passes=False),
)
def _sc_deg(dst_hbm, out_hbm, dst_v, deg_v):
    c = lax.axis_index("c")
    s = lax.axis_index("s")
    w = c * NSUB + s
    pltpu.sync_copy(dst_hbm.at[pl.ds(w * CPT, CPT)], dst_v)

    def zbody(i, carry):
        deg_v[pl.ds(i * 16, 16)] = jnp.zeros((16,), jnp.float32)
        return carry

    lax.fori_loop(0, NPAD // 16, zbody, 0)
    ones = jnp.ones((16,), jnp.float32)

    def body(i, carry):
        d = dst_v[i // 8, pl.ds((i % 8) * 16, 16)]
        plsc.addupdate_scatter(deg_v, [d], ones)
        return carry

    lax.fori_loop(0, CPT * 8, body, 0)
    pltpu.sync_copy(deg_v, out_hbm.at[pl.ds(w * NPAD, NPAD)])


@functools.partial(
    pl.kernel,
    out_type=jax.ShapeDtypeStruct((NCORES, NPAD, HIDDEN), jnp.float32),
    mesh=_MESH,
    scratch_types=[
        pltpu.VMEM((CPT, K), jnp.int32),
        pltpu.VMEM((CPT, K), jnp.int32),
        pltpu.VMEM((K, HIDDEN), jnp.float32),
        pltpu.VMEM_SHARED((NPAD, HIDDEN), jnp.float32),
        pltpu.SemaphoreType.DMA,
    ],
)
def _sc_prop(g_hbm, src_hbm, dst_hbm, zeros_hbm, out_hbm,
             src_v, dst_v, rows_v, acc, sem):
    c = lax.axis_index("c")
    s = lax.axis_index("s")
    w = c * NSUB + s
    pltpu.sync_copy(src_hbm.at[pl.ds(w * CPT, CPT)], src_v)
    pltpu.sync_copy(dst_hbm.at[pl.ds(w * CPT, CPT)], dst_v)
    pltpu.sync_copy(zeros_hbm, acc.at[pl.ds(s * ZPT, ZPT)])
    plsc.subcore_barrier()

    def body(i, carry):
        pltpu.async_copy(g_hbm.at[src_v.at[i]], rows_v, sem).wait()
        pltpu.sync_copy(rows_v, acc.at[dst_v.at[i]], add=True)
        return carry

    lax.fori_loop(0, CPT, body, 0)
    plsc.subcore_barrier()
    pltpu.sync_copy(acc.at[pl.ds(s * ZPT, ZPT)],
                    out_hbm.at[c, pl.ds(s * ZPT, ZPT)])


_RB = 1000
_GRID = N_NODES // _RB
_RB1 = 2048
_GRID1 = NPAD // _RB1


def _tc_first_body(deg_ref, x_ref, w_ref, dis_ref, g_ref):
    deg = jnp.sum(deg_ref[...], axis=0)[:, None] + 1.0
    dis = lax.rsqrt(jnp.maximum(deg, 1e-12))
    dis_ref[...] = dis
    g_ref[...] = dis * jnp.dot(x_ref[...], w_ref[...],
                               preferred_element_type=jnp.float32)


def _tc_first(deg_parts, x, W0):
    return pl.pallas_call(
        _tc_first_body,
        grid=(_GRID1,),
        in_specs=[
            pl.BlockSpec((NW, _RB1), lambda i: (0, i)),
            pl.BlockSpec((_RB1, D_FEAT), lambda i: (i, 0)),
            pl.BlockSpec((D_FEAT, HIDDEN), lambda i: (0, 0)),
        ],
        out_specs=[
            pl.BlockSpec((_RB1, 1), lambda i: (i, 0)),
            pl.BlockSpec((_RB1, HIDDEN), lambda i: (i, 0)),
        ],
        out_shape=[
            jax.ShapeDtypeStruct((N_NODES, 1), jnp.float32),
            jax.ShapeDtypeStruct((N_NODES, HIDDEN), jnp.float32),
        ],
    )(deg_parts, x, W0)


def _tc_mid_body(z_ref, g_ref, dis_ref, b_ref, w_ref, out_ref):
    dis = dis_ref[...]
    h = dis * (z_ref[0] + z_ref[1] + g_ref[...]) + b_ref[...]
    h = jnp.maximum(h, 0.0)
    out_ref[...] = dis * jnp.dot(h, w_ref[...],
                                 preferred_element_type=jnp.float32)


def _tc_mid(z, g, dis, b, W):
    return pl.pallas_call(
        _tc_mid_body,
        grid=(_GRID,),
        in_specs=[
            pl.BlockSpec((NCORES, _RB, HIDDEN), lambda i: (0, i, 0)),
            pl.BlockSpec((_RB, HIDDEN), lambda i: (i, 0)),
            pl.BlockSpec((_RB, 1), lambda i: (i, 0)),
            pl.BlockSpec((1, HIDDEN), lambda i: (0, 0)),
            pl.BlockSpec((HIDDEN, HIDDEN), lambda i: (0, 0)),
        ],
        out_specs=pl.BlockSpec((_RB, HIDDEN), lambda i: (i, 0)),
        out_shape=jax.ShapeDtypeStruct((N_NODES, HIDDEN), jnp.float32),
    )(z, g, dis, b, W)


def _tc_final_body(z_ref, g_ref, dis_ref, b_ref, batch_ref, wl_ref, bl_ref,
                   out_ref, accs, acccnt):
    i = pl.program_id(0)

    @pl.when(i == 0)
    def _():
        accs[...] = jnp.zeros_like(accs)
        acccnt[...] = jnp.zeros_like(acccnt)

    dis = dis_ref[...]
    h = dis * (z_ref[0] + z_ref[1] + g_ref[...]) + b_ref[...]
    h = jnp.maximum(h, 0.0)
    bidx = batch_ref[0, 0, :]
    gids = lax.broadcasted_iota(jnp.int32, (N_GRAPHS, _RB), 0)
    onehot = (gids == jnp.broadcast_to(bidx[None, :], (N_GRAPHS, _RB))
              ).astype(jnp.float32)
    accs[...] += jnp.dot(onehot, h, preferred_element_type=jnp.float32)
    acccnt[...] += jnp.sum(onehot, axis=1, keepdims=True)

    @pl.when(i == _GRID - 1)
    def _():
        pooled = accs[...] / jnp.maximum(acccnt[...], 1.0)
        out_ref[...] = jnp.dot(pooled, wl_ref[...],
                               preferred_element_type=jnp.float32) + bl_ref[...]


def _tc_final(z, g, dis, b, batchr, Wlin, blin):
    return pl.pallas_call(
        _tc_final_body,
        grid=(_GRID,),
        in_specs=[
            pl.BlockSpec((NCORES, _RB, HIDDEN), lambda i: (0, i, 0)),
            pl.BlockSpec((_RB, HIDDEN), lambda i: (i, 0)),
            pl.BlockSpec((_RB, 1), lambda i: (i, 0)),
            pl.BlockSpec((1, HIDDEN), lambda i: (0, 0)),
            pl.BlockSpec((1, 1, _RB), lambda i: (i, 0, 0)),
            pl.BlockSpec((HIDDEN, N_CLASSES), lambda i: (0, 0)),
            pl.BlockSpec((1, N_CLASSES), lambda i: (0, 0)),
        ],
        out_specs=pl.BlockSpec((N_GRAPHS, N_CLASSES), lambda i: (0, 0)),
        out_shape=jax.ShapeDtypeStruct((N_GRAPHS, N_CLASSES), jnp.float32),
        scratch_shapes=[
            pltpu.VMEM((N_GRAPHS, HIDDEN), jnp.float32),
            pltpu.VMEM((N_GRAPHS, 1), jnp.float32),
        ],
    )(z, g, dis, b, batchr, Wlin, blin)


def kernel(x, edge_index, batch, W0, b0, W1, b1, W2, b2, Wlin, blin):
    pad = E_PAD - N_EDGES
    src2d = jnp.concatenate(
        [edge_index[0], jnp.zeros((pad,), jnp.int32)]).reshape(CH_PAD, K)
    pad_dst = N_NODES + jnp.arange(pad, dtype=jnp.int32) % (NPAD - N_NODES)
    dst2d = jnp.concatenate(
        [edge_index[1], pad_dst]).reshape(CH_PAD, K)

    zrows = jnp.zeros((ZPT, HIDDEN), jnp.float32)
    batchr = batch.reshape(_GRID, 1, _RB)
    b0r = b0.reshape(1, HIDDEN)
    b1r = b1.reshape(1, HIDDEN)
    b2r = b2.reshape(1, HIDDEN)
    blr = blin.reshape(1, N_CLASSES)

    deg_parts = _sc_deg(dst2d).reshape(NW, NPAD)
    dis, g0 = _tc_first(deg_parts, x, W0)
    z0 = _sc_prop(g0, src2d, dst2d, zrows)
    g1 = _tc_mid(z0, g0, dis, b0r, W1)
    z1 = _sc_prop(g1, src2d, dst2d, zrows)
    g2 = _tc_mid(z1, g1, dis, b1r, W2)
    z2 = _sc_prop(g2, src2d, dst2d, zrows)
    return _tc_final(z2, g2, dis, b2r, batchr, Wlin, blr)


# speedup vs baseline: 8.6720x; 1.1398x over previous
import functools

import jax
import jax.numpy as jnp
from jax import lax
from jax.experimental import pallas as pl
from jax.experimental.pallas import tpu as pltpu
from jax.experimental.pallas import tpu_sc as plsc

N_NODES = 10000
D_FEAT = 128
HIDDEN = 128
N_GRAPHS = 64
N_CLASSES = 10
N_EDGES = 320000

NCORES = 2
NSUB = 16
NW = NCORES * NSUB
K = 128
CPT = 80
CH_PAD = CPT * NW
E_PAD = CH_PAD * K
NPAD = 10240
ZPT = NPAD // NSUB

_MESH = plsc.VectorSubcoreMesh(
    core_axis_name="c", subcore_axis_name="s", num_cores=NCORES,
    num_subcores=NSUB)


@functools.partial(
    pl.kernel,
    out_type=jax.ShapeDtypeStruct((NW * NPAD,), jnp.float32),
    mesh=_MESH,
    scratch_types=[
        pltpu.VMEM((CPT, K), jnp.int32),
        pltpu.VMEM((NPAD,), jnp.float32),
    ],
    compiler_params=pltpu.CompilerParams(needs_layout_passes=False),
)
def _sc_deg(dst_hbm, out_hbm, dst_v, deg_v):
    c = lax.axis_index("c")
    s = lax.axis_index("s")
    w = c * NSUB + s
    pltpu.sync_copy(dst_hbm.at[pl.ds(w * CPT, CPT)], dst_v)

    def zbody(i, carry):
        deg_v[pl.ds(i * 16, 16)] = jnp.zeros((16,), jnp.float32)
        return carry

    lax.fori_loop(0, NPAD // 16, zbody, 0)
    ones = jnp.ones((16,), jnp.float32)

    def body(i, carry):
        d = dst_v[i // 8, pl.ds((i % 8) * 16, 16)]
        plsc.addupdate_scatter(deg_v, [d], ones)
        return carry

    lax.fori_loop(0, CPT * 8, body, 0)
    pltpu.sync_copy(deg_v, out_hbm.at[pl.ds(w * NPAD, NPAD)])


@functools.partial(
    pl.kernel,
    out_type=jax.ShapeDtypeStruct((NCORES, NPAD, HIDDEN), jnp.float32),
    mesh=_MESH,
    scratch_types=[
        pltpu.VMEM((CPT // 2, K), jnp.int32),
        pltpu.VMEM((CPT // 2, K), jnp.int32),
        pltpu.VMEM((K, HIDDEN), jnp.float32),
        pltpu.VMEM((K, HIDDEN), jnp.float32),
        pltpu.VMEM_SHARED((NPAD, HIDDEN), jnp.float32),
        pltpu.SemaphoreType.DMA,
        pltpu.SemaphoreType.DMA,
    ],
)
def _sc_prop(g_hbm, src_hbm, dst_hbm, zeros_hbm, out_hbm,
             src_v, dst_v, rows_a, rows_b, acc, sem_a, sem_b):
    c = lax.axis_index("c")
    s = lax.axis_index("s")
    w = c * NSUB + s
    pltpu.sync_copy(zeros_hbm, acc.at[pl.ds(s * ZPT, ZPT)])
    plsc.subcore_barrier()

    HALF = CPT // 2
    for half in range(2):
        base = w * CPT + half * HALF
        pltpu.sync_copy(src_hbm.at[pl.ds(base, HALF)], src_v)
        pltpu.sync_copy(dst_hbm.at[pl.ds(base, HALF)], dst_v)
        pltpu.async_copy(g_hbm.at[src_v.at[0]], rows_a, sem_a)
        pltpu.async_copy(g_hbm.at[src_v.at[1]], rows_b, sem_b)

        def body(j, carry):
            def step(i, rows, sem):
                pltpu.make_async_copy(g_hbm.at[pl.ds(0, K)], rows, sem).wait()
                pltpu.sync_copy(rows, acc.at[dst_v.at[i]], add=True)

                @pl.when(j < HALF // 2 - 1)
                def _():
                    pltpu.async_copy(g_hbm.at[src_v.at[i + 2]], rows, sem)

            step(2 * j, rows_a, sem_a)
            step(2 * j + 1, rows_b, sem_b)
            return carry

        lax.fori_loop(0, HALF // 2, body, 0)
    plsc.subcore_barrier()
    pltpu.sync_copy(acc.at[pl.ds(s * ZPT, ZPT)],
                    out_hbm.at[c, pl.ds(s * ZPT, ZPT)])


_RB = 1000
_GRID = N_NODES // _RB
_RB1 = 2048
_GRID1 = NPAD // _RB1


def _tc_first_body(deg_ref, x_ref, w_ref, dis_ref, g_ref):
    deg = jnp.sum(deg_ref[...], axis=0)[:, None] + 1.0
    dis = lax.rsqrt(jnp.maximum(deg, 1e-12))
    dis_ref[...] = dis
    g_ref[...] = dis * jnp.dot(x_ref[...], w_ref[...],
                               preferred_element_type=jnp.float32)


def _tc_first(deg_parts, x, W0):
    return pl.pallas_call(
        _tc_first_body,
        grid=(_GRID1,),
        in_specs=[
            pl.BlockSpec((NW, _RB1), lambda i: (0, i)),
            pl.BlockSpec((_RB1, D_FEAT), lambda i: (i, 0)),
            pl.BlockSpec((D_FEAT, HIDDEN), lambda i: (0, 0)),
        ],
        out_specs=[
            pl.BlockSpec((_RB1, 1), lambda i: (i, 0)),
            pl.BlockSpec((_RB1, HIDDEN), lambda i: (i, 0)),
        ],
        out_shape=[
            jax.ShapeDtypeStruct((N_NODES, 1), jnp.float32),
            jax.ShapeDtypeStruct((N_NODES, HIDDEN), jnp.float32),
        ],
    )(deg_parts, x, W0)


def _tc_mid_body(z_ref, g_ref, dis_ref, b_ref, w_ref, out_ref):
    dis = dis_ref[...]
    h = dis * (z_ref[0] + z_ref[1] + g_ref[...]) + b_ref[...]
    h = jnp.maximum(h, 0.0)
    out_ref[...] = dis * jnp.dot(h, w_ref[...],
                                 preferred_element_type=jnp.float32)


def _tc_mid(z, g, dis, b, W):
    return pl.pallas_call(
        _tc_mid_body,
        grid=(_GRID,),
        in_specs=[
            pl.BlockSpec((NCORES, _RB, HIDDEN), lambda i: (0, i, 0)),
            pl.BlockSpec((_RB, HIDDEN), lambda i: (i, 0)),
            pl.BlockSpec((_RB, 1), lambda i: (i, 0)),
            pl.BlockSpec((1, HIDDEN), lambda i: (0, 0)),
            pl.BlockSpec((HIDDEN, HIDDEN), lambda i: (0, 0)),
        ],
        out_specs=pl.BlockSpec((_RB, HIDDEN), lambda i: (i, 0)),
        out_shape=jax.ShapeDtypeStruct((N_NODES, HIDDEN), jnp.float32),
    )(z, g, dis, b, W)


def _tc_final_body(z_ref, g_ref, dis_ref, b_ref, batch_ref, wl_ref, bl_ref,
                   out_ref, accs, acccnt):
    i = pl.program_id(0)

    @pl.when(i == 0)
    def _():
        accs[...] = jnp.zeros_like(accs)
        acccnt[...] = jnp.zeros_like(acccnt)

    dis = dis_ref[...]
    h = dis * (z_ref[0] + z_ref[1] + g_ref[...]) + b_ref[...]
    h = jnp.maximum(h, 0.0)
    bidx = batch_ref[0, 0, :]
    gids = lax.broadcasted_iota(jnp.int32, (N_GRAPHS, _RB), 0)
    onehot = (gids == jnp.broadcast_to(bidx[None, :], (N_GRAPHS, _RB))
              ).astype(jnp.float32)
    accs[...] += jnp.dot(onehot, h, preferred_element_type=jnp.float32)
    acccnt[...] += jnp.sum(onehot, axis=1, keepdims=True)

    @pl.when(i == _GRID - 1)
    def _():
        pooled = accs[...] / jnp.maximum(acccnt[...], 1.0)
        out_ref[...] = jnp.dot(pooled, wl_ref[...],
                               preferred_element_type=jnp.float32) + bl_ref[...]


def _tc_final(z, g, dis, b, batchr, Wlin, blin):
    return pl.pallas_call(
        _tc_final_body,
        grid=(_GRID,),
        in_specs=[
            pl.BlockSpec((NCORES, _RB, HIDDEN), lambda i: (0, i, 0)),
            pl.BlockSpec((_RB, HIDDEN), lambda i: (i, 0)),
            pl.BlockSpec((_RB, 1), lambda i: (i, 0)),
            pl.BlockSpec((1, HIDDEN), lambda i: (0, 0)),
            pl.BlockSpec((1, 1, _RB), lambda i: (i, 0, 0)),
            pl.BlockSpec((HIDDEN, N_CLASSES), lambda i: (0, 0)),
            pl.BlockSpec((1, N_CLASSES), lambda i: (0, 0)),
        ],
        out_specs=pl.BlockSpec((N_GRAPHS, N_CLASSES), lambda i: (0, 0)),
        out_shape=jax.ShapeDtypeStruct((N_GRAPHS, N_CLASSES), jnp.float32),
        scratch_shapes=[
            pltpu.VMEM((N_GRAPHS, HIDDEN), jnp.float32),
            pltpu.VMEM((N_GRAPHS, 1), jnp.float32),
        ],
    )(z, g, dis, b, batchr, Wlin, blin)


def kernel(x, edge_index, batch, W0, b0, W1, b1, W2, b2, Wlin, blin):
    pad = E_PAD - N_EDGES
    src2d = jnp.concatenate(
        [edge_index[0], jnp.zeros((pad,), jnp.int32)]).reshape(CH_PAD, K)
    pad_dst = N_NODES + jnp.arange(pad, dtype=jnp.int32) % (NPAD - N_NODES)
    dst2d = jnp.concatenate(
        [edge_index[1], pad_dst]).reshape(CH_PAD, K)

    zrows = jnp.zeros((ZPT, HIDDEN), jnp.float32)
    batchr = batch.reshape(_GRID, 1, _RB)
    b0r = b0.reshape(1, HIDDEN)
    b1r = b1.reshape(1, HIDDEN)
    b2r = b2.reshape(1, HIDDEN)
    blr = blin.reshape(1, N_CLASSES)

    deg_parts = _sc_deg(dst2d).reshape(NW, NPAD)
    dis, g0 = _tc_first(deg_parts, x, W0)
    z0 = _sc_prop(g0, src2d, dst2d, zrows)
    g1 = _tc_mid(z0, g0, dis, b0r, W1)
    z1 = _sc_prop(g1, src2d, dst2d, zrows)
    g2 = _tc_mid(z1, g1, dis, b1r, W2)
    z2 = _sc_prop(g2, src2d, dst2d, zrows)
    return _tc_final(z2, g2, dis, b2r, batchr, Wlin, blr)
